# Initial kernel scaffold; baseline (speedup 1.0000x reference)
import jax
import jax.numpy as jnp
from jax.experimental import pallas as pl


def kernel(x, edge_index, batch, emb_table, W1, a_src1, a_dst1, b1, W2, a_src2, a_dst2, b2, W3, a_src3, a_dst3, b3, Wl, bl):
    raise NotImplementedError("write your pallas kernel here")


# speedup vs baseline: 17.9498x; 17.9498x over previous
import functools

import jax
import jax.numpy as jnp
from jax import lax
from jax.experimental import pallas as pl
from jax.experimental.pallas import tpu as pltpu
from jax.experimental.pallas import tpu_sc as plsc

N = 50000
E = 800000
VOCAB = 10000
EMB = 64
HID = 128
HEADS = 4
OUT = HID // HEADS
G = 64
NC = 2

NCORE = 2
NSUB = 16
NW = NCORE * NSUB
CH = 128

NP_ = 50176
VP = 10240
BN = 512
NB = NP_ // BN
VB = VP // BN

CHN = 112
NPW = NP_ // NW
NCH_N = NPW // CHN

ETOT = E + N
EPAD = 851968
EPW = EPAD // NW
NCH_E = EPW // CH

MW = 40
NPS = NP_ // NSUB

BE = 2048
GE = EPAD // BE

_mesh = plsc.VectorSubcoreMesh(core_axis_name="c", subcore_axis_name="s")


def _f32(shape):
  return jax.ShapeDtypeStruct(shape, jnp.float32)


@functools.partial(
    pl.kernel,
    mesh=_mesh,
    out_type=[_f32((NP_, HID)), _f32((NP_, 8))],
    scratch_types=[
        pltpu.VMEM((CHN,), jnp.int32),
        pltpu.VMEM((CHN, HID), jnp.float32),
        pltpu.VMEM((CHN, 8), jnp.float32),
    ],
    compiler_params=pltpu.CompilerParams(use_tc_tiling_on_sc=False),
)
def _sc_node_gather(x_hbm, ew, asad_t, hw_out, asad_out, idx_v, rows_v, a_v):
  wid = lax.axis_index("s") * NCORE + lax.axis_index("c")

  @pl.loop(0, NCH_N)
  def _chunk(ch):
    base = wid * NPW + ch * CHN
    pltpu.sync_copy(x_hbm.at[pl.ds(base, CHN)], idx_v)
    pltpu.sync_copy(ew.at[idx_v], rows_v)
    pltpu.sync_copy(rows_v, hw_out.at[pl.ds(base, CHN)])
    pltpu.sync_copy(asad_t.at[idx_v], a_v)
    pltpu.sync_copy(a_v, asad_out.at[pl.ds(base, CHN)])


@functools.partial(
    pl.kernel,
    mesh=_mesh,
    out_type=[_f32((EPAD, 8)), _f32((EPAD, 8)), _f32((EPAD, HID))],
    scratch_types=[
        pltpu.VMEM((CH,), jnp.int32),
        pltpu.VMEM((CH,), jnp.int32),
        pltpu.VMEM((CH, 8), jnp.float32),
        pltpu.VMEM((CH, 8), jnp.float32),
        pltpu.VMEM((CH, HID), jnp.float32),
    ],
    compiler_params=pltpu.CompilerParams(use_tc_tiling_on_sc=False),
)
def _sc_edge_gather(src_hbm, dst_hbm, asad, hw,
                    se_out, de_out, rows_out,
                    srcv, dstv, s_v, d_v, rows_v):
  wid = lax.axis_index("s") * NCORE + lax.axis_index("c")

  @pl.loop(0, NCH_E)
  def _chunk(ch):
    base = wid * EPW + ch * CH
    pltpu.sync_copy(src_hbm.at[pl.ds(base, CH)], srcv)
    pltpu.sync_copy(dst_hbm.at[pl.ds(base, CH)], dstv)
    pltpu.sync_copy(asad.at[srcv], s_v)
    pltpu.sync_copy(s_v, se_out.at[pl.ds(base, CH)])
    pltpu.sync_copy(asad.at[dstv], d_v)
    pltpu.sync_copy(d_v, de_out.at[pl.ds(base, CH)])
    pltpu.sync_copy(hw.at[srcv], rows_v)
    pltpu.sync_copy(rows_v, rows_out.at[pl.ds(base, CH)])


@functools.partial(
    pl.kernel,
    mesh=_mesh,
    out_type=[_f32((NCORE, NP_, MW)) for _ in range(HEADS)],
    scratch_types=[
        pltpu.VMEM((CH,), jnp.int32),
        pltpu.VMEM((CH, MW), jnp.float32),
        pltpu.VMEM_SHARED((NP_, MW), jnp.float32),
    ],
    compiler_params=pltpu.CompilerParams(use_tc_tiling_on_sc=False),
)
def _sc_edge_scatter(dst_hbm, msg0, msg1, msg2, msg3, zeros_hbm,
                     acc0, acc1, acc2, acc3,
                     dstv, msgv, acc_sh):
  cid = lax.axis_index("c")
  sid = lax.axis_index("s")

  for msg, acc in ((msg0, acc0), (msg1, acc1), (msg2, acc2), (msg3, acc3)):
    pltpu.sync_copy(zeros_hbm, acc_sh.at[pl.ds(sid * NPS, NPS)])
    plsc.subcore_barrier()

    @pl.loop(0, NCH_E)
    def _chunk(ch):
      base = (sid * NCORE + cid) * EPW + ch * CH
      pltpu.sync_copy(dst_hbm.at[pl.ds(base, CH)], dstv)
      pltpu.sync_copy(msg.at[pl.ds(base, CH)], msgv)
      pltpu.sync_copy(msgv, acc_sh.at[dstv], add=True)

    plsc.subcore_barrier()
    pltpu.sync_copy(acc_sh.at[pl.ds(sid * NPS, NPS)],
                    acc.at[cid, pl.ds(sid * NPS, NPS)])
    plsc.subcore_barrier()


def _tc_vocab_body(emb_ref, w_ref, am_ref, ew_out, asad_out):
  ew = jnp.dot(emb_ref[...], w_ref[...], preferred_element_type=jnp.float32)
  ew_out[...] = ew
  asad_out[...] = jnp.dot(ew, am_ref[...], preferred_element_type=jnp.float32)


def _tc_vocab(emb_pad, W1, asadm):
  return pl.pallas_call(
      _tc_vocab_body,
      grid=(VB,),
      in_specs=[
          pl.BlockSpec((BN, EMB), lambda i: (i, 0)),
          pl.BlockSpec((EMB, HID), lambda i: (0, 0)),
          pl.BlockSpec((HID, 8), lambda i: (0, 0)),
      ],
      out_specs=[pl.BlockSpec((BN, HID), lambda i: (i, 0)),
                 pl.BlockSpec((BN, 8), lambda i: (i, 0))],
      out_shape=[_f32((VP, HID)), _f32((VP, 8))],
  )(emb_pad, W1, asadm)


def _tc_edge_body(se_ref, de_ref, rows_ref, *outs):
  v = se_ref[...][:, :HEADS] + de_ref[...][:, HEADS:]
  w = jnp.exp(jnp.maximum(v, 0.2 * v))
  rows = rows_ref[...]
  pad = jnp.zeros((BE, MW - OUT - HEADS), jnp.float32)
  for h in range(HEADS):
    msg = rows[:, h * OUT:(h + 1) * OUT] * w[:, h:h + 1]
    outs[h][...] = jnp.concatenate([msg, w, pad], axis=1)


def _tc_edge(se, de, rows):
  return pl.pallas_call(
      _tc_edge_body,
      grid=(GE,),
      in_specs=[
          pl.BlockSpec((BE, 8), lambda i: (i, 0)),
          pl.BlockSpec((BE, 8), lambda i: (i, 0)),
          pl.BlockSpec((BE, HID), lambda i: (i, 0)),
      ],
      out_specs=[pl.BlockSpec((BE, MW), lambda i: (i, 0))
                 for _ in range(HEADS)],
      out_shape=[_f32((EPAD, MW)) for _ in range(HEADS)],
  )(se, de, rows)


def _node_features(a_refs, b_ref):
  parts = []
  for h in range(HEADS):
    acc = a_refs[h][0] + a_refs[h][1]
    den = acc[:, OUT + h:OUT + h + 1] + 1e-16
    parts.append(acc[:, :OUT] / den)
  hcat = jnp.concatenate(parts, axis=1)
  return jnp.maximum(hcat + b_ref[...], 0.0)


def _tc_prep_body(a0, a1, a2, a3, b_ref, w_ref, am_ref, hw_out, asad_out):
  hnode = _node_features((a0, a1, a2, a3), b_ref)
  hw = jnp.dot(hnode, w_ref[...], preferred_element_type=jnp.float32)
  hw_out[...] = hw
  asad_out[...] = jnp.dot(hw, am_ref[...], preferred_element_type=jnp.float32)


def _tc_prep(accs, b, W, asadm):
  return pl.pallas_call(
      _tc_prep_body,
      grid=(NB,),
      in_specs=[pl.BlockSpec((NCORE, BN, MW), lambda i: (0, i, 0))
                for _ in range(HEADS)] + [
          pl.BlockSpec((1, HID), lambda i: (0, 0)),
          pl.BlockSpec((HID, HID), lambda i: (0, 0)),
          pl.BlockSpec((HID, 8), lambda i: (0, 0)),
      ],
      out_specs=[pl.BlockSpec((BN, HID), lambda i: (i, 0)),
                 pl.BlockSpec((BN, 8), lambda i: (i, 0))],
      out_shape=[_f32((NP_, HID)), _f32((NP_, 8))],
  )(*accs, b, W, asadm)


def _tc_pool_body(a0, a1, a2, a3, b_ref, batch_ref, wl_ref, bl_ref,
                  out_ref, seg_ref, cnt_ref):
  i = pl.program_id(0)
  hnode = _node_features((a0, a1, a2, a3), b_ref)
  giota = lax.broadcasted_iota(jnp.int32, (BN, G), 1)
  onehot = (batch_ref[...] == giota).astype(jnp.float32)
  dn = (((0,), (0,)), ((), ()))
  segpart = lax.dot_general(onehot, hnode, dn,
                            preferred_element_type=jnp.float32)
  cntpart = lax.dot_general(onehot, jnp.ones((BN, HID), jnp.float32), dn,
                            preferred_element_type=jnp.float32)

  @pl.when(i == 0)
  def _():
    seg_ref[...] = jnp.zeros_like(seg_ref)
    cnt_ref[...] = jnp.zeros_like(cnt_ref)

  seg_ref[...] += segpart
  cnt_ref[...] += cntpart

  @pl.when(i == NB - 1)
  def _():
    pooled = seg_ref[...] / jnp.maximum(cnt_ref[...], 1.0)
    out_ref[...] = jnp.dot(pooled, wl_ref[...],
                           preferred_element_type=jnp.float32) + bl_ref[...]


def _tc_pool(accs, b, batch2d, wlp, blp):
  return pl.pallas_call(
      _tc_pool_body,
      grid=(NB,),
      in_specs=[pl.BlockSpec((NCORE, BN, MW), lambda i: (0, i, 0))
                for _ in range(HEADS)] + [
          pl.BlockSpec((1, HID), lambda i: (0, 0)),
          pl.BlockSpec((BN, 1), lambda i: (i, 0)),
          pl.BlockSpec((HID, HID), lambda i: (0, 0)),
          pl.BlockSpec((1, HID), lambda i: (0, 0)),
      ],
      out_specs=pl.BlockSpec((G, HID), lambda i: (0, 0)),
      out_shape=_f32((G, HID)),
      scratch_shapes=[pltpu.VMEM((G, HID), jnp.float32),
                      pltpu.VMEM((G, HID), jnp.float32)],
  )(*accs, b, batch2d, wlp, blp)


def _asadm(a_src, a_dst):
  mask = (jnp.arange(HID)[:, None] // OUT == jnp.arange(HEADS)[None, :])
  mask = mask.astype(jnp.float32)
  asm = mask * a_src.reshape(HID)[:, None]
  adm = mask * a_dst.reshape(HID)[:, None]
  return jnp.concatenate([asm, adm], axis=1)


def kernel(x, edge_index, batch, emb_table,
           W1, a_src1, a_dst1, b1, W2, a_src2, a_dst2, b2,
           W3, a_src3, a_dst3, b3, Wl, bl):
  i32 = jnp.int32
  x_pad = jnp.concatenate([x.astype(i32), jnp.zeros((NP_ - N,), i32)])
  loops = jnp.arange(N, dtype=i32)
  npad = EPAD - ETOT
  pad_fill = jnp.arange(npad, dtype=i32) % 64
  src = jnp.concatenate([edge_index[0].astype(i32), loops, pad_fill])
  dst = jnp.concatenate([edge_index[1].astype(i32), loops, N + pad_fill])
  batch2d = jnp.concatenate(
      [batch.astype(i32), jnp.full((NP_ - N,), G, i32)]).reshape(NP_, 1)
  emb_pad = jnp.concatenate(
      [emb_table, jnp.zeros((VP - VOCAB, EMB), jnp.float32)])
  wlp = jnp.zeros((HID, HID), jnp.float32).at[:, :NC].set(Wl)
  blp = jnp.zeros((1, HID), jnp.float32).at[0, :NC].set(bl)
  zeros_sh = jnp.zeros((NPS, MW), jnp.float32)

  ew, asad_t = _tc_vocab(emb_pad, W1, _asadm(a_src1, a_dst1))
  hw, asad = _sc_node_gather(x_pad, ew, asad_t)

  accs = None
  for li, (W, a_s, a_d, b) in enumerate((
      (None, None, None, b1), (W2, a_src2, a_dst2, b2),
      (W3, a_src3, a_dst3, b3))):
    if li > 0:
      hw, asad = _tc_prep(accs, b_prev.reshape(1, HID), W, _asadm(a_s, a_d))
    se, de, rows = _sc_edge_gather(src, dst, asad, hw)
    msgs = _tc_edge(se, de, rows)
    accs = _sc_edge_scatter(dst, *msgs, zeros_sh)
    b_prev = b

  out = _tc_pool(accs, b3.reshape(1, HID), batch2d, wlp, blp)
  return out[:, :NC]


# speedup vs baseline: 20.1553x; 1.1229x over previous
import functools

import jax
import jax.numpy as jnp
from jax import lax
from jax.experimental import pallas as pl
from jax.experimental.pallas import tpu as pltpu
from jax.experimental.pallas import tpu_sc as plsc

N = 50000
E = 800000
VOCAB = 10000
EMB = 64
HID = 128
HEADS = 4
OUT = HID // HEADS
G = 64
NC = 2

NCORE = 2
NSUB = 16
NW = NCORE * NSUB
CH = 128

NP_ = 50176
VP = 10240
BN = 512
NB = NP_ // BN
VB = VP // BN

CHN = 112
NPW = NP_ // NW
NCH_N = NPW // CHN

ETOT = E + N
EPAD = 851968
EPW = EPAD // NW
NCH_E = EPW // CH

MW = 40
NPS = NP_ // NSUB

BE = 2048
GE = EPAD // BE

_mesh = plsc.VectorSubcoreMesh(core_axis_name="c", subcore_axis_name="s")


def _f32(shape):
  return jax.ShapeDtypeStruct(shape, jnp.float32)


@functools.partial(
    pl.kernel,
    mesh=_mesh,
    out_type=[_f32((NP_, HID)), _f32((NP_, 8))],
    scratch_types=[
        pltpu.VMEM((CHN,), jnp.int32),
        pltpu.VMEM((CHN, HID), jnp.float32),
        pltpu.VMEM((CHN, 8), jnp.float32),
    ],
    compiler_params=pltpu.CompilerParams(use_tc_tiling_on_sc=False),
)
def _sc_node_gather(x_hbm, ew, asad_t, hw_out, asad_out, idx_v, rows_v, a_v):
  wid = lax.axis_index("s") * NCORE + lax.axis_index("c")

  @pl.loop(0, NCH_N)
  def _chunk(ch):
    base = wid * NPW + ch * CHN
    pltpu.sync_copy(x_hbm.at[pl.ds(base, CHN)], idx_v)
    pltpu.sync_copy(ew.at[idx_v], rows_v)
    pltpu.sync_copy(rows_v, hw_out.at[pl.ds(base, CHN)])
    pltpu.sync_copy(asad_t.at[idx_v], a_v)
    pltpu.sync_copy(a_v, asad_out.at[pl.ds(base, CHN)])


@functools.partial(
    pl.kernel,
    mesh=_mesh,
    out_type=[_f32((EPAD, 8)), _f32((EPAD, 8)), _f32((EPAD, HID))],
    scratch_types=[
        pltpu.VMEM((CH,), jnp.int32), pltpu.VMEM((CH,), jnp.int32),
        pltpu.VMEM((CH,), jnp.int32), pltpu.VMEM((CH,), jnp.int32),
        pltpu.VMEM((CH, 8), jnp.float32), pltpu.VMEM((CH, 8), jnp.float32),
        pltpu.VMEM((CH, 8), jnp.float32), pltpu.VMEM((CH, 8), jnp.float32),
        pltpu.VMEM((CH, HID), jnp.float32), pltpu.VMEM((CH, HID), jnp.float32),
        pltpu.SemaphoreType.DMA, pltpu.SemaphoreType.DMA,
        pltpu.SemaphoreType.DMA, pltpu.SemaphoreType.DMA,
        pltpu.SemaphoreType.DMA, pltpu.SemaphoreType.DMA,
    ],
    compiler_params=pltpu.CompilerParams(use_tc_tiling_on_sc=False),
)
def _sc_edge_gather(src_hbm, dst_hbm, asad, hw,
                    se_out, de_out, rows_out,
                    srcv0, srcv1, dstv0, dstv1, sv0, sv1, dv0, dv1, rv0, rv1,
                    si0, si1, sg0, sg1, ss0, ss1):
  wid = lax.axis_index("s") * NCORE + lax.axis_index("c")
  bufs = ((srcv0, dstv0, sv0, dv0, rv0, si0, sg0, ss0),
          (srcv1, dstv1, sv1, dv1, rv1, si1, sg1, ss1))

  def idx_issue(p, c):
    srcv, dstv, _, _, _, si, _, _ = bufs[p]
    base = wid * EPW + c * CH
    pltpu.async_copy(src_hbm.at[pl.ds(base, CH)], srcv, si)
    pltpu.async_copy(dst_hbm.at[pl.ds(base, CH)], dstv, si)

  def idx_wait(p):
    srcv, dstv, _, _, _, si, _, _ = bufs[p]
    pltpu.make_async_copy(src_hbm.at[pl.ds(0, CH)], srcv, si).wait()
    pltpu.make_async_copy(dst_hbm.at[pl.ds(0, CH)], dstv, si).wait()

  def gat_issue(p):
    srcv, dstv, s_v, d_v, rows_v, _, sg, _ = bufs[p]
    pltpu.async_copy(asad.at[srcv], s_v, sg)
    pltpu.async_copy(asad.at[dstv], d_v, sg)
    pltpu.async_copy(hw.at[srcv], rows_v, sg)

  def gat_wait(p):
    srcv, dstv, s_v, d_v, rows_v, _, sg, _ = bufs[p]
    pltpu.make_async_copy(asad.at[srcv], s_v, sg).wait()
    pltpu.make_async_copy(asad.at[dstv], d_v, sg).wait()
    pltpu.make_async_copy(hw.at[srcv], rows_v, sg).wait()

  def st_issue(p, c):
    _, _, s_v, d_v, rows_v, _, _, ss = bufs[p]
    base = wid * EPW + c * CH
    pltpu.async_copy(s_v, se_out.at[pl.ds(base, CH)], ss)
    pltpu.async_copy(d_v, de_out.at[pl.ds(base, CH)], ss)
    pltpu.async_copy(rows_v, rows_out.at[pl.ds(base, CH)], ss)

  def st_wait(p):
    _, _, s_v, d_v, rows_v, _, _, ss = bufs[p]
    pltpu.make_async_copy(s_v, se_out.at[pl.ds(0, CH)], ss).wait()
    pltpu.make_async_copy(d_v, de_out.at[pl.ds(0, CH)], ss).wait()
    pltpu.make_async_copy(rows_v, rows_out.at[pl.ds(0, CH)], ss).wait()

  idx_issue(0, 0)
  idx_issue(1, 1)

  @pl.loop(0, NCH_E // 2)
  def _i(i):
    for p in (0, 1):
      @pl.when(i > 0)
      def _():
        st_wait(p)
      idx_wait(p)
      gat_issue(p)
    for p in (0, 1):
      c = 2 * i + p
      gat_wait(p)
      st_issue(p, c)

      @pl.when(i < NCH_E // 2 - 1)
      def _():
        idx_issue(p, c + 2)

  st_wait(0)
  st_wait(1)


@functools.partial(
    pl.kernel,
    mesh=_mesh,
    out_type=[_f32((NCORE, NP_, MW)) for _ in range(HEADS)],
    scratch_types=[
        pltpu.VMEM((CH,), jnp.int32),
        pltpu.VMEM((CH, MW), jnp.float32),
        pltpu.VMEM_SHARED((NP_, MW), jnp.float32),
    ],
    compiler_params=pltpu.CompilerParams(use_tc_tiling_on_sc=False),
)
def _sc_edge_scatter(dst_hbm, msg0, msg1, msg2, msg3, zeros_hbm,
                     acc0, acc1, acc2, acc3,
                     dstv, msgv, acc_sh):
  cid = lax.axis_index("c")
  sid = lax.axis_index("s")
  wid = sid * NCORE + cid

  for msg, acc in ((msg0, acc0), (msg1, acc1), (msg2, acc2), (msg3, acc3)):
    pltpu.sync_copy(zeros_hbm, acc_sh.at[pl.ds(sid * NPS, NPS)])
    plsc.subcore_barrier()

    @pl.loop(0, NCH_E)
    def _chunk(ch):
      base = wid * EPW + ch * CH
      pltpu.sync_copy(dst_hbm.at[pl.ds(base, CH)], dstv)
      pltpu.sync_copy(msg.at[pl.ds(base, CH)], msgv)
      pltpu.sync_copy(msgv, acc_sh.at[dstv], add=True)

    plsc.subcore_barrier()
    pltpu.sync_copy(acc_sh.at[pl.ds(sid * NPS, NPS)],
                    acc.at[cid, pl.ds(sid * NPS, NPS)])
    plsc.subcore_barrier()


def _tc_vocab_body(emb_ref, w_ref, am_ref, ew_out, asad_out):
  ew = jnp.dot(emb_ref[...], w_ref[...], preferred_element_type=jnp.float32)
  ew_out[...] = ew
  asad_out[...] = jnp.dot(ew, am_ref[...], preferred_element_type=jnp.float32)


def _tc_vocab(emb_pad, W1, asadm):
  return pl.pallas_call(
      _tc_vocab_body,
      grid=(VB,),
      in_specs=[
          pl.BlockSpec((BN, EMB), lambda i: (i, 0)),
          pl.BlockSpec((EMB, HID), lambda i: (0, 0)),
          pl.BlockSpec((HID, 8), lambda i: (0, 0)),
      ],
      out_specs=[pl.BlockSpec((BN, HID), lambda i: (i, 0)),
                 pl.BlockSpec((BN, 8), lambda i: (i, 0))],
      out_shape=[_f32((VP, HID)), _f32((VP, 8))],
  )(emb_pad, W1, asadm)


def _tc_edge_body(se_ref, de_ref, rows_ref, *outs):
  v = se_ref[...][:, :HEADS] + de_ref[...][:, HEADS:]
  w = jnp.exp(jnp.maximum(v, 0.2 * v))
  rows = rows_ref[...]
  pad = jnp.zeros((BE, MW - OUT - HEADS), jnp.float32)
  for h in range(HEADS):
    msg = rows[:, h * OUT:(h + 1) * OUT] * w[:, h:h + 1]
    outs[h][...] = jnp.concatenate([msg, w, pad], axis=1)


def _tc_edge(se, de, rows):
  return pl.pallas_call(
      _tc_edge_body,
      grid=(GE,),
      in_specs=[
          pl.BlockSpec((BE, 8), lambda i: (i, 0)),
          pl.BlockSpec((BE, 8), lambda i: (i, 0)),
          pl.BlockSpec((BE, HID), lambda i: (i, 0)),
      ],
      out_specs=[pl.BlockSpec((BE, MW), lambda i: (i, 0))
                 for _ in range(HEADS)],
      out_shape=[_f32((EPAD, MW)) for _ in range(HEADS)],
  )(se, de, rows)


def _node_features(a_refs, b_ref):
  parts = []
  for h in range(HEADS):
    acc = a_refs[h][0] + a_refs[h][1]
    den = acc[:, OUT + h:OUT + h + 1] + 1e-16
    parts.append(acc[:, :OUT] / den)
  hcat = jnp.concatenate(parts, axis=1)
  return jnp.maximum(hcat + b_ref[...], 0.0)


def _tc_prep_body(a0, a1, a2, a3, b_ref, w_ref, am_ref, hw_out, asad_out):
  hnode = _node_features((a0, a1, a2, a3), b_ref)
  hw = jnp.dot(hnode, w_ref[...], preferred_element_type=jnp.float32)
  hw_out[...] = hw
  asad_out[...] = jnp.dot(hw, am_ref[...], preferred_element_type=jnp.float32)


def _tc_prep(accs, b, W, asadm):
  return pl.pallas_call(
      _tc_prep_body,
      grid=(NB,),
      in_specs=[pl.BlockSpec((NCORE, BN, MW), lambda i: (0, i, 0))
                for _ in range(HEADS)] + [
          pl.BlockSpec((1, HID), lambda i: (0, 0)),
          pl.BlockSpec((HID, HID), lambda i: (0, 0)),
          pl.BlockSpec((HID, 8), lambda i: (0, 0)),
      ],
      out_specs=[pl.BlockSpec((BN, HID), lambda i: (i, 0)),
                 pl.BlockSpec((BN, 8), lambda i: (i, 0))],
      out_shape=[_f32((NP_, HID)), _f32((NP_, 8))],
  )(*accs, b, W, asadm)


def _tc_pool_body(a0, a1, a2, a3, b_ref, batch_ref, wl_ref, bl_ref,
                  out_ref, seg_ref, cnt_ref):
  i = pl.program_id(0)
  hnode = _node_features((a0, a1, a2, a3), b_ref)
  giota = lax.broadcasted_iota(jnp.int32, (BN, G), 1)
  onehot = (batch_ref[...] == giota).astype(jnp.float32)
  dn = (((0,), (0,)), ((), ()))
  segpart = lax.dot_general(onehot, hnode, dn,
                            preferred_element_type=jnp.float32)
  cntpart = lax.dot_general(onehot, jnp.ones((BN, HID), jnp.float32), dn,
                            preferred_element_type=jnp.float32)

  @pl.when(i == 0)
  def _():
    seg_ref[...] = jnp.zeros_like(seg_ref)
    cnt_ref[...] = jnp.zeros_like(cnt_ref)

  seg_ref[...] += segpart
  cnt_ref[...] += cntpart

  @pl.when(i == NB - 1)
  def _():
    pooled = seg_ref[...] / jnp.maximum(cnt_ref[...], 1.0)
    out_ref[...] = jnp.dot(pooled, wl_ref[...],
                           preferred_element_type=jnp.float32) + bl_ref[...]


def _tc_pool(accs, b, batch2d, wlp, blp):
  return pl.pallas_call(
      _tc_pool_body,
      grid=(NB,),
      in_specs=[pl.BlockSpec((NCORE, BN, MW), lambda i: (0, i, 0))
                for _ in range(HEADS)] + [
          pl.BlockSpec((1, HID), lambda i: (0, 0)),
          pl.BlockSpec((BN, 1), lambda i: (i, 0)),
          pl.BlockSpec((HID, HID), lambda i: (0, 0)),
          pl.BlockSpec((1, HID), lambda i: (0, 0)),
      ],
      out_specs=pl.BlockSpec((G, HID), lambda i: (0, 0)),
      out_shape=_f32((G, HID)),
      scratch_shapes=[pltpu.VMEM((G, HID), jnp.float32),
                      pltpu.VMEM((G, HID), jnp.float32)],
  )(*accs, b, batch2d, wlp, blp)


def _asadm(a_src, a_dst):
  mask = (jnp.arange(HID)[:, None] // OUT == jnp.arange(HEADS)[None, :])
  mask = mask.astype(jnp.float32)
  asm = mask * a_src.reshape(HID)[:, None]
  adm = mask * a_dst.reshape(HID)[:, None]
  return jnp.concatenate([asm, adm], axis=1)


def kernel(x, edge_index, batch, emb_table,
           W1, a_src1, a_dst1, b1, W2, a_src2, a_dst2, b2,
           W3, a_src3, a_dst3, b3, Wl, bl):
  i32 = jnp.int32
  x_pad = jnp.concatenate([x.astype(i32), jnp.zeros((NP_ - N,), i32)])
  loops = jnp.arange(N, dtype=i32)
  npad = EPAD - ETOT
  pad_fill = jnp.arange(npad, dtype=i32) % 64
  src = jnp.concatenate([edge_index[0].astype(i32), loops, pad_fill])
  dst = jnp.concatenate([edge_index[1].astype(i32), loops, N + pad_fill])
  batch2d = jnp.concatenate(
      [batch.astype(i32), jnp.full((NP_ - N,), G, i32)]).reshape(NP_, 1)
  emb_pad = jnp.concatenate(
      [emb_table, jnp.zeros((VP - VOCAB, EMB), jnp.float32)])
  wlp = jnp.zeros((HID, HID), jnp.float32).at[:, :NC].set(Wl)
  blp = jnp.zeros((1, HID), jnp.float32).at[0, :NC].set(bl)
  zeros_sh = jnp.zeros((NPS, MW), jnp.float32)

  ew, asad_t = _tc_vocab(emb_pad, W1, _asadm(a_src1, a_dst1))
  hw, asad = _sc_node_gather(x_pad, ew, asad_t)

  accs = None
  for li, (W, a_s, a_d, b) in enumerate((
      (None, None, None, b1), (W2, a_src2, a_dst2, b2),
      (W3, a_src3, a_dst3, b3))):
    if li > 0:
      hw, asad = _tc_prep(accs, b_prev.reshape(1, HID), W, _asadm(a_s, a_d))
    se, de, rows = _sc_edge_gather(src, dst, asad, hw)
    msgs = _tc_edge(se, de, rows)
    accs = _sc_edge_scatter(dst, *msgs, zeros_sh)
    b_prev = b

  out = _tc_pool(accs, b3.reshape(1, HID), batch2d, wlp, blp)
  return out[:, :NC]
